# baseline (speedup 1.0000x reference)
import jax
import jax.numpy as jnp
from jax.experimental import pallas as pl
from jax.experimental.pallas import tpu as pltpu

_VMEM_LIMIT = 52 * 1024 * 1024


def _prep_1x1(w, scale):
    return (w[:, :, 0, 0].T * scale[None, :]).astype(jnp.bfloat16)


def _prep_3x3(w, scale):
    cout, cin = w.shape[0], w.shape[1]
    wt = jnp.transpose(w, (2, 3, 1, 0)).reshape(3, 3 * cin, cout)
    return (wt * scale[None, None, :]).astype(jnp.bfloat16)


def _prep_bias(shift, c):
    return shift.reshape(1, c).astype(jnp.float32)


def _prep_conv1(w, scale):
    wt = jnp.transpose(w, (2, 3, 1, 0)) * scale[None, None, None, :]
    wt = wt.reshape(7, 21, 64)
    cols = [jnp.pad(wt, ((0, 0), (6 * wl, 107 - 6 * wl), (0, 0)))
            for wl in range(16)]
    g = jnp.stack(cols, axis=2).reshape(7, 128, 1024)
    return g.astype(jnp.bfloat16)


def _conv1_pool_body(x_ref, p_ref, g_ref, b_ref, o_ref):
    xc = x_ref[...].reshape(3, 256, 128).astype(jnp.bfloat16)
    s = jnp.concatenate([xc[0], xc[1], xc[2],
                         jnp.zeros((256, 128), jnp.bfloat16)], axis=1)
    a = jnp.dot(s, p_ref[...], preferred_element_type=jnp.float32)
    a = a.astype(jnp.bfloat16)
    a = jnp.pad(a, ((3, 3), (0, 0)))
    a = a.reshape(131, 2, 402)
    a0 = a[:, 0:1, :].reshape(131, 402)
    a1 = a[:, 1:2, :].reshape(131, 402)
    par = (a0, a1)
    z14 = jnp.zeros((128, 14), jnp.bfloat16)
    groups = []
    for g in range(4):
        acc = None
        for kh in range(7):
            t, o = kh // 2, kh % 2
            if g < 3:
                b = par[o][t:t + 128, 96 * g:96 * g + 128]
            else:
                b = jnp.concatenate([par[o][t:t + 128, 288:402], z14], axis=1)
            d = jnp.dot(b, g_ref[kh], preferred_element_type=jnp.float32)
            acc = d if acc is None else acc + d
        groups.append(jnp.maximum(acc + b_ref[...], 0.0).astype(jnp.bfloat16))
    c = jnp.concatenate(groups, axis=1)
    z64 = jnp.zeros((128, 64), jnp.bfloat16)
    t1 = jnp.maximum(c, jnp.concatenate([c[:, 64:], z64], axis=1))
    t1 = jnp.maximum(t1, jnp.concatenate([z64, c[:, :4032]], axis=1))
    t2 = jnp.concatenate([t1[:, 128 * wp:128 * wp + 64] for wp in range(32)],
                         axis=1)
    t2 = jnp.pad(t2, ((1, 1), (0, 0)))
    hr = t2.reshape(65, 2, 2048)
    he = hr[:, 0:1, :].reshape(65, 2048)
    ho = hr[:, 1:2, :].reshape(65, 2048)
    m = jnp.maximum(jnp.maximum(he[0:64], ho[0:64]), he[1:65])
    for wp in range(32):
        o_ref[0, :, wp, :] = m[:, 64 * wp:64 * wp + 64]


def _conv1_pool(x, p, g, b, n):
    out = pl.pallas_call(
        _conv1_pool_body,
        out_shape=jax.ShapeDtypeStruct((n, 64, 32, 64), jnp.bfloat16),
        grid=(n,),
        in_specs=[
            pl.BlockSpec((1, 3, 256, 128), lambda i: (i, 0, 0, 0)),
            pl.BlockSpec((512, 402), lambda i: (0, 0)),
            pl.BlockSpec((7, 128, 1024), lambda i: (0, 0, 0)),
            pl.BlockSpec((1, 1024), lambda i: (0, 0)),
        ],
        out_specs=pl.BlockSpec((1, 64, 32, 64), lambda i: (i, 0, 0, 0)),
        compiler_params=pltpu.CompilerParams(
            dimension_semantics=("parallel",),
            vmem_limit_bytes=_VMEM_LIMIT),
    )(x, p, g, b)
    return out.reshape(n * 2048, 64)


def _bottleneck_body(bg, h, w, cin, p, cout, stride, has_down):
    ho, wo = h // stride, w // stride
    mo = bg * ho * wo

    def body(*refs):
        if has_down:
            x_ref, w1_ref, b1_ref, w2_ref, b2_ref, w3_ref, b3_ref, \
                wd_ref, bd_ref, o_ref = refs
        else:
            x_ref, w1_ref, b1_ref, w2_ref, b2_ref, w3_ref, b3_ref, o_ref = refs

        x = x_ref[...]
        y1 = jnp.dot(x, w1_ref[...], preferred_element_type=jnp.float32)
        y1 = jnp.maximum(y1 + b1_ref[...], 0.0).astype(jnp.bfloat16)
        y1 = y1.reshape(bg, h, w, p)
        y1 = jnp.pad(y1, ((0, 0), (1, 1), (1, 1), (0, 0)))

        acc = None
        if stride == 1:
            for kh in range(3):
                taps = [y1[:, kh:kh + h, kw:kw + w, :].reshape(mo, p)
                        for kw in range(3)]
                d = jnp.dot(jnp.concatenate(taps, axis=1), w2_ref[kh],
                            preferred_element_type=jnp.float32)
                acc = d if acc is None else acc + d
        else:
            hp, wp2 = (h + 2) // 2, (w + 2) // 2
            yr = y1.reshape(bg, hp, 2, wp2, 2, p)
            par = {}
            for oh in (0, 1):
                for ow in (0, 1):
                    par[(oh, ow)] = yr[:, :, oh:oh + 1, :, ow:ow + 1, :] \
                        .reshape(bg, hp, wp2, p)
            for th, oh in ((0, 0), (0, 1), (1, 0)):
                taps = []
                for tw, ow in ((0, 0), (0, 1), (1, 0)):
                    tap = par[(oh, ow)][:, th:th + ho, tw:tw + wo, :]
                    taps.append(tap.reshape(mo, p))
                d = jnp.dot(jnp.concatenate(taps, axis=1), w2_ref[2 * th + oh],
                            preferred_element_type=jnp.float32)
                acc = d if acc is None else acc + d
        y2 = jnp.maximum(acc + b2_ref[...], 0.0).astype(jnp.bfloat16)

        acc3 = jnp.dot(y2, w3_ref[...], preferred_element_type=jnp.float32)
        acc3 = acc3 + b3_ref[...]
        if has_down:
            if stride == 2:
                xs = x.reshape(bg, h // 2, 2, w // 2, 2, cin)
                xs = xs[:, :, 0:1, :, 0:1, :].reshape(mo, cin)
            else:
                xs = x
            res = jnp.dot(xs, wd_ref[...], preferred_element_type=jnp.float32)
            res = (res + bd_ref[...]).astype(jnp.bfloat16)
            acc3 = acc3 + res.astype(jnp.float32)
        else:
            acc3 = acc3 + x.astype(jnp.float32)
        o_ref[...] = jnp.maximum(acc3, 0.0).astype(jnp.bfloat16)

    return body


def _bottleneck(x2d, wp, n, h, w, cin, p, cout, stride, bg):
    ho, wo = h // stride, w // stride
    has_down = "wd" in wp
    m_in, m_out = bg * h * w, bg * ho * wo

    args = [x2d, wp["w1"], wp["b1"], wp["w2"], wp["b2"], wp["w3"], wp["b3"]]
    in_specs = [
        pl.BlockSpec((m_in, cin), lambda i: (i, 0)),
        pl.BlockSpec((cin, p), lambda i: (0, 0)),
        pl.BlockSpec((1, p), lambda i: (0, 0)),
        pl.BlockSpec((3, 3 * p, p), lambda i: (0, 0, 0)),
        pl.BlockSpec((1, p), lambda i: (0, 0)),
        pl.BlockSpec((p, cout), lambda i: (0, 0)),
        pl.BlockSpec((1, cout), lambda i: (0, 0)),
    ]
    if has_down:
        args += [wp["wd"], wp["bd"]]
        in_specs += [
            pl.BlockSpec((cin, cout), lambda i: (0, 0)),
            pl.BlockSpec((1, cout), lambda i: (0, 0)),
        ]

    return pl.pallas_call(
        _bottleneck_body(bg, h, w, cin, p, cout, stride, has_down),
        out_shape=jax.ShapeDtypeStruct((n * ho * wo, cout), jnp.bfloat16),
        grid=(n // bg,),
        in_specs=in_specs,
        out_specs=pl.BlockSpec((m_out, cout), lambda i: (i, 0)),
        compiler_params=pltpu.CompilerParams(
            dimension_semantics=("parallel",),
            vmem_limit_bytes=_VMEM_LIMIT),
    )(*args)


def _head_body(n):
    def body(x_ref, w_ref, b_ref, o_ref):
        x = x_ref[...].reshape(n, 32, 2048)
        feats = jnp.mean(x.astype(jnp.float32), axis=1).astype(jnp.bfloat16)
        out = jnp.dot(feats, w_ref[...], preferred_element_type=jnp.float32)
        o_ref[...] = out + b_ref[...]
    return body


def _head(x2d, fc_w, fc_b, n, nclass):
    return pl.pallas_call(
        _head_body(n),
        out_shape=jax.ShapeDtypeStruct((n, nclass), jnp.float32),
        grid=(2,),
        in_specs=[
            pl.BlockSpec((n * 32, 2048), lambda j: (0, 0)),
            pl.BlockSpec((2048, nclass // 2), lambda j: (0, j)),
            pl.BlockSpec((1, nclass // 2), lambda j: (0, j)),
        ],
        out_specs=pl.BlockSpec((n, nclass // 2), lambda j: (0, j)),
        compiler_params=pltpu.CompilerParams(
            dimension_semantics=("parallel",),
            vmem_limit_bytes=_VMEM_LIMIT),
    )(x2d, fc_w.astype(jnp.bfloat16), fc_b.reshape(1, nclass).astype(jnp.float32))


_CFG = ((64, 3, 1), (128, 4, 2), (256, 6, 2), (512, 3, 2))
_BG = (2, 2, 4, 8)


def kernel(x, conv1_w, conv1_scale, conv1_shift, l0b0_conv1_w, l0b0_conv1_scale, l0b0_conv1_shift, l0b0_conv2_w, l0b0_conv2_scale, l0b0_conv2_shift, l0b0_conv3_w, l0b0_conv3_scale, l0b0_conv3_shift, l0b0_down_w, l0b0_down_scale, l0b0_down_shift, l0b1_conv1_w, l0b1_conv1_scale, l0b1_conv1_shift, l0b1_conv2_w, l0b1_conv2_scale, l0b1_conv2_shift, l0b1_conv3_w, l0b1_conv3_scale, l0b1_conv3_shift, l0b2_conv1_w, l0b2_conv1_scale, l0b2_conv1_shift, l0b2_conv2_w, l0b2_conv2_scale, l0b2_conv2_shift, l0b2_conv3_w, l0b2_conv3_scale, l0b2_conv3_shift, l1b0_conv1_w, l1b0_conv1_scale, l1b0_conv1_shift, l1b0_conv2_w, l1b0_conv2_scale, l1b0_conv2_shift, l1b0_conv3_w, l1b0_conv3_scale, l1b0_conv3_shift, l1b0_down_w, l1b0_down_scale, l1b0_down_shift, l1b1_conv1_w, l1b1_conv1_scale, l1b1_conv1_shift, l1b1_conv2_w, l1b1_conv2_scale, l1b1_conv2_shift, l1b1_conv3_w, l1b1_conv3_scale, l1b1_conv3_shift, l1b2_conv1_w, l1b2_conv1_scale, l1b2_conv1_shift, l1b2_conv2_w, l1b2_conv2_scale, l1b2_conv2_shift, l1b2_conv3_w, l1b2_conv3_scale, l1b2_conv3_shift, l1b3_conv1_w, l1b3_conv1_scale, l1b3_conv1_shift, l1b3_conv2_w, l1b3_conv2_scale, l1b3_conv2_shift, l1b3_conv3_w, l1b3_conv3_scale, l1b3_conv3_shift, l2b0_conv1_w, l2b0_conv1_scale, l2b0_conv1_shift, l2b0_conv2_w, l2b0_conv2_scale, l2b0_conv2_shift, l2b0_conv3_w, l2b0_conv3_scale, l2b0_conv3_shift, l2b0_down_w, l2b0_down_scale, l2b0_down_shift, l2b1_conv1_w, l2b1_conv1_scale, l2b1_conv1_shift, l2b1_conv2_w, l2b1_conv2_scale, l2b1_conv2_shift, l2b1_conv3_w, l2b1_conv3_scale, l2b1_conv3_shift, l2b2_conv1_w, l2b2_conv1_scale, l2b2_conv1_shift, l2b2_conv2_w, l2b2_conv2_scale, l2b2_conv2_shift, l2b2_conv3_w, l2b2_conv3_scale, l2b2_conv3_shift, l2b3_conv1_w, l2b3_conv1_scale, l2b3_conv1_shift, l2b3_conv2_w, l2b3_conv2_scale, l2b3_conv2_shift, l2b3_conv3_w, l2b3_conv3_scale, l2b3_conv3_shift, l2b4_conv1_w, l2b4_conv1_scale, l2b4_conv1_shift, l2b4_conv2_w, l2b4_conv2_scale, l2b4_conv2_shift, l2b4_conv3_w, l2b4_conv3_scale, l2b4_conv3_shift, l2b5_conv1_w, l2b5_conv1_scale, l2b5_conv1_shift, l2b5_conv2_w, l2b5_conv2_scale, l2b5_conv2_shift, l2b5_conv3_w, l2b5_conv3_scale, l2b5_conv3_shift, l3b0_conv1_w, l3b0_conv1_scale, l3b0_conv1_shift, l3b0_conv2_w, l3b0_conv2_scale, l3b0_conv2_shift, l3b0_conv3_w, l3b0_conv3_scale, l3b0_conv3_shift, l3b0_down_w, l3b0_down_scale, l3b0_down_shift, l3b1_conv1_w, l3b1_conv1_scale, l3b1_conv1_shift, l3b1_conv2_w, l3b1_conv2_scale, l3b1_conv2_shift, l3b1_conv3_w, l3b1_conv3_scale, l3b1_conv3_shift, l3b2_conv1_w, l3b2_conv1_scale, l3b2_conv1_shift, l3b2_conv2_w, l3b2_conv2_scale, l3b2_conv2_shift, l3b2_conv3_w, l3b2_conv3_scale, l3b2_conv3_shift, fc_w, fc_b):
    d = locals()
    n = x.shape[0]

    j = jnp.arange(402)
    v, c = j // 3, j % 3
    perm = jnp.where((v >= 3) & (v < 131), c * 128 + v - 3, 384)
    p1 = (jnp.arange(512)[:, None] == perm[None, :]).astype(jnp.bfloat16)
    b1 = jnp.tile(conv1_shift.astype(jnp.float32), 16).reshape(1, 1024)
    y = _conv1_pool(x, p1, _prep_conv1(conv1_w, conv1_scale), b1, n)

    h, w, cin = 64, 32, 64
    for li, (planes, nblocks, stride) in enumerate(_CFG):
        cout = planes * 4
        for b in range(nblocks):
            s = stride if b == 0 else 1
            pre = f"l{li}b{b}_"
            wp = {
                "w1": _prep_1x1(d[pre + "conv1_w"], d[pre + "conv1_scale"]),
                "b1": _prep_bias(d[pre + "conv1_shift"], planes),
                "w2": _prep_3x3(d[pre + "conv2_w"], d[pre + "conv2_scale"]),
                "b2": _prep_bias(d[pre + "conv2_shift"], planes),
                "w3": _prep_1x1(d[pre + "conv3_w"], d[pre + "conv3_scale"]),
                "b3": _prep_bias(d[pre + "conv3_shift"], cout),
            }
            if pre + "down_w" in d:
                wp["wd"] = _prep_1x1(d[pre + "down_w"], d[pre + "down_scale"])
                wp["bd"] = _prep_bias(d[pre + "down_shift"], cout)
            y = _bottleneck(y, wp, n, h, w, cin, planes, cout, s,
                            min(_BG[li], n))
            h, w, cin = h // s, w // s, cout

    return _head(y, fc_w, fc_b, n, fc_b.shape[0])


# speedup vs baseline: 1.0139x; 1.0139x over previous
import jax
import jax.numpy as jnp
from jax.experimental import pallas as pl
from jax.experimental.pallas import tpu as pltpu

_VMEM_LIMIT = 52 * 1024 * 1024


def _prep_1x1(w, scale):
    return (w[:, :, 0, 0].T * scale[None, :]).astype(jnp.bfloat16)


def _prep_3x3(w, scale):
    cout, cin = w.shape[0], w.shape[1]
    wt = jnp.transpose(w, (2, 3, 1, 0)).reshape(3, 3 * cin, cout)
    return (wt * scale[None, None, :]).astype(jnp.bfloat16)


def _prep_bias(shift, c):
    return shift.reshape(1, c).astype(jnp.float32)


def _prep_conv1(w, scale):
    wt = jnp.transpose(w, (2, 3, 1, 0)) * scale[None, None, None, :]
    wt = wt.reshape(7, 21, 64)
    cols = [jnp.pad(wt, ((0, 0), (6 * wl, 107 - 6 * wl), (0, 0)))
            for wl in range(16)]
    g = jnp.stack(cols, axis=2).reshape(7, 128, 1024)
    return g.astype(jnp.bfloat16)


def _conv1_pool_body(x_ref, p_ref, g_ref, b_ref, o_ref):
    xc = x_ref[...].reshape(3, 256, 128).astype(jnp.bfloat16)
    s = jnp.concatenate([xc[0], xc[1], xc[2],
                         jnp.zeros((256, 128), jnp.bfloat16)], axis=1)
    a = jnp.dot(s, p_ref[...], preferred_element_type=jnp.float32)
    a = a.astype(jnp.bfloat16)
    a = jnp.pad(a, ((3, 3), (0, 0)))
    a = a.reshape(131, 2, 402)
    a0 = a[:, 0:1, :].reshape(131, 402)
    a1 = a[:, 1:2, :].reshape(131, 402)
    par = (a0, a1)
    z14 = jnp.zeros((128, 14), jnp.bfloat16)
    groups = []
    for g in range(4):
        acc = None
        for kh in range(7):
            t, o = kh // 2, kh % 2
            if g < 3:
                b = par[o][t:t + 128, 96 * g:96 * g + 128]
            else:
                b = jnp.concatenate([par[o][t:t + 128, 288:402], z14], axis=1)
            d = jnp.dot(b, g_ref[kh], preferred_element_type=jnp.float32)
            acc = d if acc is None else acc + d
        groups.append(jnp.maximum(acc + b_ref[...], 0.0).astype(jnp.bfloat16))
    c = jnp.concatenate(groups, axis=1)
    z64 = jnp.zeros((128, 64), jnp.bfloat16)
    t1 = jnp.maximum(c, jnp.concatenate([c[:, 64:], z64], axis=1))
    t1 = jnp.maximum(t1, jnp.concatenate([z64, c[:, :4032]], axis=1))
    t2 = jnp.concatenate([t1[:, 128 * wp:128 * wp + 64] for wp in range(32)],
                         axis=1)
    t2 = jnp.pad(t2, ((1, 1), (0, 0)))
    hr = t2.reshape(65, 2, 2048)
    he = hr[:, 0:1, :].reshape(65, 2048)
    ho = hr[:, 1:2, :].reshape(65, 2048)
    m = jnp.maximum(jnp.maximum(he[0:64], ho[0:64]), he[1:65])
    for wp in range(32):
        o_ref[0, :, wp, :] = m[:, 64 * wp:64 * wp + 64]


def _conv1_pool(x, p, g, b, n):
    out = pl.pallas_call(
        _conv1_pool_body,
        out_shape=jax.ShapeDtypeStruct((n, 64, 32, 64), jnp.bfloat16),
        grid=(n,),
        in_specs=[
            pl.BlockSpec((1, 3, 256, 128), lambda i: (i, 0, 0, 0)),
            pl.BlockSpec((512, 402), lambda i: (0, 0)),
            pl.BlockSpec((7, 128, 1024), lambda i: (0, 0, 0)),
            pl.BlockSpec((1, 1024), lambda i: (0, 0)),
        ],
        out_specs=pl.BlockSpec((1, 64, 32, 64), lambda i: (i, 0, 0, 0)),
        compiler_params=pltpu.CompilerParams(
            dimension_semantics=("parallel",),
            vmem_limit_bytes=_VMEM_LIMIT),
    )(x, p, g, b)
    return out.reshape(n * 2048, 64)


def _bottleneck_body(bg, h, w, cin, p, cout, stride, has_down):
    ho, wo = h // stride, w // stride
    mo = bg * ho * wo

    def body(*refs):
        if has_down:
            x_ref, w1_ref, b1_ref, w2_ref, b2_ref, w3_ref, b3_ref, \
                wd_ref, bd_ref, o_ref = refs
        else:
            x_ref, w1_ref, b1_ref, w2_ref, b2_ref, w3_ref, b3_ref, o_ref = refs

        x = x_ref[...]
        y1 = jnp.dot(x, w1_ref[...], preferred_element_type=jnp.float32)
        y1 = jnp.maximum(y1 + b1_ref[...], 0.0).astype(jnp.bfloat16)
        y1 = y1.reshape(bg, h, w, p)
        y1 = jnp.pad(y1, ((0, 0), (1, 1), (1, 1), (0, 0)))

        acc = None
        if stride == 1:
            for kh in range(3):
                taps = [y1[:, kh:kh + h, kw:kw + w, :].reshape(mo, p)
                        for kw in range(3)]
                d = jnp.dot(jnp.concatenate(taps, axis=1), w2_ref[kh],
                            preferred_element_type=jnp.float32)
                acc = d if acc is None else acc + d
        else:
            hp, wp2 = (h + 2) // 2, (w + 2) // 2
            yr = y1.reshape(bg, hp, 2, wp2, 2, p)
            par = {}
            for oh in (0, 1):
                for ow in (0, 1):
                    par[(oh, ow)] = yr[:, :, oh:oh + 1, :, ow:ow + 1, :] \
                        .reshape(bg, hp, wp2, p)
            for th, oh in ((0, 0), (0, 1), (1, 0)):
                taps = []
                for tw, ow in ((0, 0), (0, 1), (1, 0)):
                    tap = par[(oh, ow)][:, th:th + ho, tw:tw + wo, :]
                    taps.append(tap.reshape(mo, p))
                d = jnp.dot(jnp.concatenate(taps, axis=1), w2_ref[2 * th + oh],
                            preferred_element_type=jnp.float32)
                acc = d if acc is None else acc + d
        y2 = jnp.maximum(acc + b2_ref[...], 0.0).astype(jnp.bfloat16)

        acc3 = jnp.dot(y2, w3_ref[...], preferred_element_type=jnp.float32)
        acc3 = acc3 + b3_ref[...]
        if has_down:
            if stride == 2:
                xs = x.reshape(bg, h // 2, 2, w // 2, 2, cin)
                xs = xs[:, :, 0:1, :, 0:1, :].reshape(mo, cin)
            else:
                xs = x
            res = jnp.dot(xs, wd_ref[...], preferred_element_type=jnp.float32)
            res = (res + bd_ref[...]).astype(jnp.bfloat16)
            acc3 = acc3 + res.astype(jnp.float32)
        else:
            acc3 = acc3 + x.astype(jnp.float32)
        o_ref[...] = jnp.maximum(acc3, 0.0).astype(jnp.bfloat16)

    return body


def _bottleneck(x2d, wp, n, h, w, cin, p, cout, stride, bg):
    ho, wo = h // stride, w // stride
    has_down = "wd" in wp
    m_in, m_out = bg * h * w, bg * ho * wo

    args = [x2d, wp["w1"], wp["b1"], wp["w2"], wp["b2"], wp["w3"], wp["b3"]]
    in_specs = [
        pl.BlockSpec((m_in, cin), lambda i: (i, 0)),
        pl.BlockSpec((cin, p), lambda i: (0, 0)),
        pl.BlockSpec((1, p), lambda i: (0, 0)),
        pl.BlockSpec((3, 3 * p, p), lambda i: (0, 0, 0)),
        pl.BlockSpec((1, p), lambda i: (0, 0)),
        pl.BlockSpec((p, cout), lambda i: (0, 0)),
        pl.BlockSpec((1, cout), lambda i: (0, 0)),
    ]
    if has_down:
        args += [wp["wd"], wp["bd"]]
        in_specs += [
            pl.BlockSpec((cin, cout), lambda i: (0, 0)),
            pl.BlockSpec((1, cout), lambda i: (0, 0)),
        ]

    return pl.pallas_call(
        _bottleneck_body(bg, h, w, cin, p, cout, stride, has_down),
        out_shape=jax.ShapeDtypeStruct((n * ho * wo, cout), jnp.bfloat16),
        grid=(n // bg,),
        in_specs=in_specs,
        out_specs=pl.BlockSpec((m_out, cout), lambda i: (i, 0)),
        compiler_params=pltpu.CompilerParams(
            dimension_semantics=("parallel",),
            vmem_limit_bytes=_VMEM_LIMIT),
    )(*args)


def _head_body(n):
    def body(x_ref, w_ref, b_ref, o_ref):
        x = x_ref[...].reshape(n, 32, 2048)
        feats = jnp.mean(x.astype(jnp.float32), axis=1).astype(jnp.bfloat16)
        out = jnp.dot(feats, w_ref[...], preferred_element_type=jnp.float32)
        o_ref[...] = out + b_ref[...]
    return body


def _head(x2d, fc_w, fc_b, n, nclass):
    return pl.pallas_call(
        _head_body(n),
        out_shape=jax.ShapeDtypeStruct((n, nclass), jnp.float32),
        grid=(2,),
        in_specs=[
            pl.BlockSpec((n * 32, 2048), lambda j: (0, 0)),
            pl.BlockSpec((2048, nclass // 2), lambda j: (0, j)),
            pl.BlockSpec((1, nclass // 2), lambda j: (0, j)),
        ],
        out_specs=pl.BlockSpec((n, nclass // 2), lambda j: (0, j)),
        compiler_params=pltpu.CompilerParams(
            dimension_semantics=("parallel",),
            vmem_limit_bytes=_VMEM_LIMIT),
    )(x2d, fc_w.astype(jnp.bfloat16), fc_b.reshape(1, nclass).astype(jnp.float32))


_CFG = ((64, 3, 1), (128, 4, 2), (256, 6, 2), (512, 3, 2))
_BG = (4, 4, 4, 8)


def kernel(x, conv1_w, conv1_scale, conv1_shift, l0b0_conv1_w, l0b0_conv1_scale, l0b0_conv1_shift, l0b0_conv2_w, l0b0_conv2_scale, l0b0_conv2_shift, l0b0_conv3_w, l0b0_conv3_scale, l0b0_conv3_shift, l0b0_down_w, l0b0_down_scale, l0b0_down_shift, l0b1_conv1_w, l0b1_conv1_scale, l0b1_conv1_shift, l0b1_conv2_w, l0b1_conv2_scale, l0b1_conv2_shift, l0b1_conv3_w, l0b1_conv3_scale, l0b1_conv3_shift, l0b2_conv1_w, l0b2_conv1_scale, l0b2_conv1_shift, l0b2_conv2_w, l0b2_conv2_scale, l0b2_conv2_shift, l0b2_conv3_w, l0b2_conv3_scale, l0b2_conv3_shift, l1b0_conv1_w, l1b0_conv1_scale, l1b0_conv1_shift, l1b0_conv2_w, l1b0_conv2_scale, l1b0_conv2_shift, l1b0_conv3_w, l1b0_conv3_scale, l1b0_conv3_shift, l1b0_down_w, l1b0_down_scale, l1b0_down_shift, l1b1_conv1_w, l1b1_conv1_scale, l1b1_conv1_shift, l1b1_conv2_w, l1b1_conv2_scale, l1b1_conv2_shift, l1b1_conv3_w, l1b1_conv3_scale, l1b1_conv3_shift, l1b2_conv1_w, l1b2_conv1_scale, l1b2_conv1_shift, l1b2_conv2_w, l1b2_conv2_scale, l1b2_conv2_shift, l1b2_conv3_w, l1b2_conv3_scale, l1b2_conv3_shift, l1b3_conv1_w, l1b3_conv1_scale, l1b3_conv1_shift, l1b3_conv2_w, l1b3_conv2_scale, l1b3_conv2_shift, l1b3_conv3_w, l1b3_conv3_scale, l1b3_conv3_shift, l2b0_conv1_w, l2b0_conv1_scale, l2b0_conv1_shift, l2b0_conv2_w, l2b0_conv2_scale, l2b0_conv2_shift, l2b0_conv3_w, l2b0_conv3_scale, l2b0_conv3_shift, l2b0_down_w, l2b0_down_scale, l2b0_down_shift, l2b1_conv1_w, l2b1_conv1_scale, l2b1_conv1_shift, l2b1_conv2_w, l2b1_conv2_scale, l2b1_conv2_shift, l2b1_conv3_w, l2b1_conv3_scale, l2b1_conv3_shift, l2b2_conv1_w, l2b2_conv1_scale, l2b2_conv1_shift, l2b2_conv2_w, l2b2_conv2_scale, l2b2_conv2_shift, l2b2_conv3_w, l2b2_conv3_scale, l2b2_conv3_shift, l2b3_conv1_w, l2b3_conv1_scale, l2b3_conv1_shift, l2b3_conv2_w, l2b3_conv2_scale, l2b3_conv2_shift, l2b3_conv3_w, l2b3_conv3_scale, l2b3_conv3_shift, l2b4_conv1_w, l2b4_conv1_scale, l2b4_conv1_shift, l2b4_conv2_w, l2b4_conv2_scale, l2b4_conv2_shift, l2b4_conv3_w, l2b4_conv3_scale, l2b4_conv3_shift, l2b5_conv1_w, l2b5_conv1_scale, l2b5_conv1_shift, l2b5_conv2_w, l2b5_conv2_scale, l2b5_conv2_shift, l2b5_conv3_w, l2b5_conv3_scale, l2b5_conv3_shift, l3b0_conv1_w, l3b0_conv1_scale, l3b0_conv1_shift, l3b0_conv2_w, l3b0_conv2_scale, l3b0_conv2_shift, l3b0_conv3_w, l3b0_conv3_scale, l3b0_conv3_shift, l3b0_down_w, l3b0_down_scale, l3b0_down_shift, l3b1_conv1_w, l3b1_conv1_scale, l3b1_conv1_shift, l3b1_conv2_w, l3b1_conv2_scale, l3b1_conv2_shift, l3b1_conv3_w, l3b1_conv3_scale, l3b1_conv3_shift, l3b2_conv1_w, l3b2_conv1_scale, l3b2_conv1_shift, l3b2_conv2_w, l3b2_conv2_scale, l3b2_conv2_shift, l3b2_conv3_w, l3b2_conv3_scale, l3b2_conv3_shift, fc_w, fc_b):
    d = locals()
    n = x.shape[0]

    j = jnp.arange(402)
    v, c = j // 3, j % 3
    perm = jnp.where((v >= 3) & (v < 131), c * 128 + v - 3, 384)
    p1 = (jnp.arange(512)[:, None] == perm[None, :]).astype(jnp.bfloat16)
    b1 = jnp.tile(conv1_shift.astype(jnp.float32), 16).reshape(1, 1024)
    y = _conv1_pool(x, p1, _prep_conv1(conv1_w, conv1_scale), b1, n)

    h, w, cin = 64, 32, 64
    for li, (planes, nblocks, stride) in enumerate(_CFG):
        cout = planes * 4
        for b in range(nblocks):
            s = stride if b == 0 else 1
            pre = f"l{li}b{b}_"
            wp = {
                "w1": _prep_1x1(d[pre + "conv1_w"], d[pre + "conv1_scale"]),
                "b1": _prep_bias(d[pre + "conv1_shift"], planes),
                "w2": _prep_3x3(d[pre + "conv2_w"], d[pre + "conv2_scale"]),
                "b2": _prep_bias(d[pre + "conv2_shift"], planes),
                "w3": _prep_1x1(d[pre + "conv3_w"], d[pre + "conv3_scale"]),
                "b3": _prep_bias(d[pre + "conv3_shift"], cout),
            }
            if pre + "down_w" in d:
                wp["wd"] = _prep_1x1(d[pre + "down_w"], d[pre + "down_scale"])
                wp["bd"] = _prep_bias(d[pre + "down_shift"], cout)
            y = _bottleneck(y, wp, n, h, w, cin, planes, cout, s,
                            min(_BG[li], n))
            h, w, cin = h // s, w // s, cout

    return _head(y, fc_w, fc_b, n, fc_b.shape[0])
